# baseline (speedup 1.0000x reference)
import functools

import jax
import jax.numpy as jnp
from jax import lax
from jax.experimental import pallas as pl
from jax.experimental.pallas import tpu as pltpu
from jax.experimental.pallas import tpu_sc as plsc

N = 10000
E = 320000
D = 128
NG = 64
DIM_EMB = 128
DIM_OUT = 16

CHUNK = 128
TILES = 16
SLICE = 624
TAIL = N - TILES * SLICE
DEGW = 16
PAIRS_PER_TILE = -(-E // (CHUNK * TILES * 2))
CHUNKS_PER_TILE = 2 * PAIRS_PER_TILE
NUM_CHUNKS_PAD = CHUNKS_PER_TILE * TILES
E_PAD = NUM_CHUNKS_PAD * CHUNK
ACC_ROWS = N + 8


def _sc_aggregate(x0, eidx0, x1, eidx1, z128, z16, ones16):
    mesh = plsc.VectorSubcoreMesh(core_axis_name="c", subcore_axis_name="s")

    @functools.partial(
        pl.kernel,
        mesh=mesh,
        compiler_params=pltpu.CompilerParams(use_tc_tiling_on_sc=False),
        out_type=[
            jax.ShapeDtypeStruct((N, D), jnp.float32),
            jax.ShapeDtypeStruct((N, DEGW), jnp.float32),
            jax.ShapeDtypeStruct((N, D), jnp.float32),
            jax.ShapeDtypeStruct((N, DEGW), jnp.float32),
        ],
        scratch_types=[
            pltpu.VMEM((2, CHUNK), jnp.int32),
            pltpu.VMEM((2, CHUNK), jnp.int32),
            pltpu.VMEM((CHUNK, D), jnp.float32),
            pltpu.VMEM((CHUNK, D), jnp.float32),
            pltpu.VMEM((CHUNK, DEGW), jnp.float32),
            pltpu.VMEM_SHARED((ACC_ROWS, D), jnp.float32),
            pltpu.VMEM_SHARED((ACC_ROWS, DEGW), jnp.float32),
            pltpu.SemaphoreType.DMA,
            pltpu.SemaphoreType.DMA,
            pltpu.SemaphoreType.DMA,
        ],
    )
    def sc_kernel(x0_h, src0_h, dst0_h, x1_h, src1_h, dst1_h,
                  z128_h, z16_h, ones_h,
                  agg0_h, deg0_h, agg1_h, deg1_h,
                  src_v, dst_v, rows_a, rows_b, ones_v,
                  acc_sh, deg_sh, sem_a, sem_b, sem_s):
        cid = lax.axis_index("c")
        sid = lax.axis_index("s")
        row0 = sid * SLICE
        pieces = []
        off = 0
        while off < SLICE:
            sz = min(CHUNK, SLICE - off)
            pieces.append((off, sz))
            off += sz

        def run_branch(x_h, src_h, dst_h, agg_h, deg_h):
            pltpu.sync_copy(z128_h.at[pl.ds(0, CHUNK)], rows_a)
            pltpu.sync_copy(z16_h.at[pl.ds(0, CHUNK)], ones_v)
            for (o, sz) in pieces:
                pltpu.sync_copy(rows_a.at[pl.ds(0, sz)],
                                acc_sh.at[pl.ds(row0 + o, sz)])
                pltpu.sync_copy(ones_v.at[pl.ds(0, sz)],
                                deg_sh.at[pl.ds(row0 + o, sz)])

            @pl.when(sid == TILES - 1)
            def _():
                pltpu.sync_copy(rows_a.at[pl.ds(0, TAIL + 8)],
                                acc_sh.at[pl.ds(TILES * SLICE, TAIL + 8)])
                pltpu.sync_copy(ones_v.at[pl.ds(0, TAIL + 8)],
                                deg_sh.at[pl.ds(TILES * SLICE, TAIL + 8)])

            pltpu.sync_copy(ones_h, ones_v)
            plsc.subcore_barrier()

            chunk0 = sid * CHUNKS_PER_TILE

            def body(i, carry):
                c0 = chunk0 + 2 * i
                pltpu.sync_copy(src_h.at[pl.ds(c0, 2)], src_v)
                pltpu.sync_copy(dst_h.at[pl.ds(c0, 2)], dst_v)
                ga = pltpu.async_copy(x_h.at[src_v.at[0]], rows_a, sem_a)
                gb = pltpu.async_copy(x_h.at[src_v.at[1]], rows_b, sem_b)
                ga.wait()
                s0 = pltpu.async_copy(rows_a, acc_sh.at[dst_v.at[0]],
                                      sem_s, add=True)
                s1 = pltpu.async_copy(ones_v, deg_sh.at[dst_v.at[0]],
                                      sem_s, add=True)
                gb.wait()
                s2 = pltpu.async_copy(rows_b, acc_sh.at[dst_v.at[1]],
                                      sem_s, add=True)
                s3 = pltpu.async_copy(ones_v, deg_sh.at[dst_v.at[1]],
                                      sem_s, add=True)
                s0.wait()
                s1.wait()
                s2.wait()
                s3.wait()
                return carry

            lax.fori_loop(0, PAIRS_PER_TILE, body, 0)
            plsc.subcore_barrier()
            for (o, sz) in pieces:
                pltpu.sync_copy(acc_sh.at[pl.ds(row0 + o, sz)],
                                rows_a.at[pl.ds(0, sz)])
                pltpu.sync_copy(rows_a.at[pl.ds(0, sz)],
                                agg_h.at[pl.ds(row0 + o, sz)])
                pltpu.sync_copy(deg_sh.at[pl.ds(row0 + o, sz)],
                                ones_v.at[pl.ds(0, sz)])
                pltpu.sync_copy(ones_v.at[pl.ds(0, sz)],
                                deg_h.at[pl.ds(row0 + o, sz)])

            @pl.when(sid == TILES - 1)
            def _():
                pltpu.sync_copy(acc_sh.at[pl.ds(TILES * SLICE, TAIL)],
                                rows_a.at[pl.ds(0, TAIL)])
                pltpu.sync_copy(rows_a.at[pl.ds(0, TAIL)],
                                agg_h.at[pl.ds(TILES * SLICE, TAIL)])
                pltpu.sync_copy(deg_sh.at[pl.ds(TILES * SLICE, TAIL)],
                                ones_v.at[pl.ds(0, TAIL)])
                pltpu.sync_copy(ones_v.at[pl.ds(0, TAIL)],
                                deg_h.at[pl.ds(TILES * SLICE, TAIL)])

        @pl.when(cid == 0)
        def _():
            run_branch(x0_h, src0_h, dst0_h, agg0_h, deg0_h)

        @pl.when(cid == 1)
        def _():
            run_branch(x1_h, src1_h, dst1_h, agg1_h, deg1_h)

    return sc_kernel(x0, eidx0[0], eidx0[1], x1, eidx1[0], eidx1[1],
                     z128, z16, ones16)


BLK = 1000
NUM_BLK = N // BLK


def _tc_body(agg0_r, deg0_r, b0m_r, agg1_r, deg1_r, b1m_r,
             W0_r, bb0_r, W1_r, bb1_r, f1a_r, f1b_r, f1bias_r, f2w_r, f2b_r,
             out_r, h_r, acc0, cnt0, acc1, cnt1):
    k = pl.program_id(0)

    @pl.when(k == 0)
    def _():
        acc0[...] = jnp.zeros_like(acc0)
        cnt0[...] = jnp.zeros_like(cnt0)
        acc1[...] = jnp.zeros_like(acc1)
        cnt1[...] = jnp.zeros_like(cnt1)

    ones_col = jnp.ones((BLK, 1), jnp.float32)
    gids = lax.broadcasted_iota(jnp.int32, (BLK, NG), 1)

    def branch(agg_r, deg_r, bm_r, W_r, bias_r, acc, cnt):
        deg = jnp.max(deg_r[...], axis=1, keepdims=True)
        inv = 1.0 / jnp.maximum(deg, 1.0)
        h = jnp.maximum(
            jnp.dot(agg_r[...] * inv, W_r[...],
                    preferred_element_type=jnp.float32) + bias_r[...],
            0.0)
        m = (bm_r[...] == gids).astype(jnp.float32)
        acc[...] += lax.dot_general(m, h, (((0,), (0,)), ((), ())),
                                    preferred_element_type=jnp.float32)
        cnt[...] += lax.dot_general(m, ones_col, (((0,), (0,)), ((), ())),
                                    preferred_element_type=jnp.float32)

    branch(agg0_r, deg0_r, b0m_r, W0_r, bb0_r, acc0, cnt0)
    branch(agg1_r, deg1_r, b1m_r, W1_r, bb1_r, acc1, cnt1)

    @pl.when(k == NUM_BLK - 1)
    def _():
        g0 = acc0[...] / jnp.maximum(cnt0[...], 1.0)
        g1 = acc1[...] / jnp.maximum(cnt1[...], 1.0)
        hh = jnp.maximum(
            jnp.dot(g0, f1a_r[...], preferred_element_type=jnp.float32)
            + jnp.dot(g1, f1b_r[...], preferred_element_type=jnp.float32)
            + f1bias_r[...], 0.0)
        h_r[...] = hh
        out_r[...] = jnp.dot(hh, f2w_r[...],
                             preferred_element_type=jnp.float32) + f2b_r[...]


def _tc_head(agg0, deg0, batch0, agg1, deg1, batch1,
             W0, b0, W1, b1, f1a, f1b, f1bias, f2w, f2b):
    row_spec = lambda shape: pl.BlockSpec((BLK,) + shape[1:],
                                          lambda k: (k,) + (0,) * (len(shape) - 1))
    full = lambda shape: pl.BlockSpec(shape, lambda k: (0,) * len(shape))
    return pl.pallas_call(
        _tc_body,
        grid=(NUM_BLK,),
        in_specs=[
            row_spec((N, D)), row_spec((N, DEGW)), row_spec((N, 1)),
            row_spec((N, D)), row_spec((N, DEGW)), row_spec((N, 1)),
            full((D, DIM_EMB)), full((1, DIM_EMB)),
            full((D, DIM_EMB)), full((1, DIM_EMB)),
            full((DIM_EMB, 64)), full((DIM_EMB, 64)), full((1, 64)),
            full((64, DIM_OUT)), full((1, DIM_OUT)),
        ],
        out_specs=[full((NG, DIM_OUT)), full((NG, 64))],
        out_shape=[jax.ShapeDtypeStruct((NG, DIM_OUT), jnp.float32),
                   jax.ShapeDtypeStruct((NG, 64), jnp.float32)],
        scratch_shapes=[
            pltpu.VMEM((NG, DIM_EMB), jnp.float32),
            pltpu.VMEM((NG, 1), jnp.float32),
            pltpu.VMEM((NG, DIM_EMB), jnp.float32),
            pltpu.VMEM((NG, 1), jnp.float32),
        ],
    )(agg0, deg0, batch0, agg1, deg1, batch1,
      W0, b0, W1, b1, f1a, f1b, f1bias, f2w, f2b)


def kernel(x0, edge_index0, batch0, x1, edge_index1, batch1,
           W0, b0, W1, b1, fc1_W, fc1_b, fc2_W, fc2_b):
    z128 = jnp.zeros((SLICE, D), jnp.float32)
    z16 = jnp.zeros((SLICE, DEGW), jnp.float32)
    ones16 = jnp.ones((CHUNK, DEGW), jnp.float32)

    def prep(ei):
        srcp = jnp.concatenate([ei[0], jnp.zeros((E_PAD - E,), jnp.int32)])
        dstp = jnp.concatenate([ei[1], jnp.full((E_PAD - E,), N, jnp.int32)])
        return (srcp.reshape(NUM_CHUNKS_PAD, CHUNK),
                dstp.reshape(NUM_CHUNKS_PAD, CHUNK))

    agg0, deg0, agg1, deg1 = _sc_aggregate(
        x0, prep(edge_index0), x1, prep(edge_index1), z128, z16, ones16)

    out, h = _tc_head(
        agg0, deg0, batch0[:, None], agg1, deg1, batch1[:, None],
        W0, b0[None, :], W1, b1[None, :],
        fc1_W[:DIM_EMB], fc1_W[DIM_EMB:], fc1_b[None, :],
        fc2_W, fc2_b[None, :])
    return (out, h)


# speedup vs baseline: 1.0837x; 1.0837x over previous
import functools

import jax
import jax.numpy as jnp
from jax import lax
from jax.experimental import pallas as pl
from jax.experimental.pallas import tpu as pltpu
from jax.experimental.pallas import tpu_sc as plsc

N = 10000
E = 320000
D = 128
NG = 64
DIM_EMB = 128
DIM_OUT = 16

CHUNK = 128
NUM_CHUNKS = E // CHUNK
TILES = 16
CHUNKS_PER_TILE = (NUM_CHUNKS + TILES - 1) // TILES
SLICE = 624
TAIL = N - TILES * SLICE
DEGW = 16


def _sc_aggregate(x0, ei0, x1, ei1, z128, z16, ones16):
    mesh = plsc.VectorSubcoreMesh(core_axis_name="c", subcore_axis_name="s")

    @functools.partial(
        pl.kernel,
        mesh=mesh,
        compiler_params=pltpu.CompilerParams(use_tc_tiling_on_sc=False),
        out_type=[
            jax.ShapeDtypeStruct((N, D), jnp.float32),
            jax.ShapeDtypeStruct((N, DEGW), jnp.float32),
            jax.ShapeDtypeStruct((N, D), jnp.float32),
            jax.ShapeDtypeStruct((N, DEGW), jnp.float32),
        ],
        scratch_types=[
            pltpu.VMEM((CHUNK,), jnp.int32),
            pltpu.VMEM((CHUNK,), jnp.int32),
            pltpu.VMEM((CHUNK, D), jnp.float32),
            pltpu.VMEM((CHUNK, DEGW), jnp.float32),
            pltpu.VMEM((SLICE, DEGW), jnp.float32),
            pltpu.VMEM_SHARED((N, D), jnp.float32),
            pltpu.VMEM_SHARED((N, DEGW), jnp.float32),
            pltpu.SemaphoreType.DMA,
        ],
    )
    def sc_kernel(x0_h, ei0_h, x1_h, ei1_h,
                  z128_h, z16_h, ones_h,
                  agg0_h, deg0_h, agg1_h, deg1_h,
                  src_v, dst_v, rows_v, ones_v, dstage_v, acc_sh, deg_sh, sem):
        cid = lax.axis_index("c")
        sid = lax.axis_index("s")
        row0 = sid * SLICE
        pieces = []
        off = 0
        while off < SLICE:
            sz = min(CHUNK, SLICE - off)
            pieces.append((off, sz))
            off += sz

        def run_branch(x_h, ei_h, agg_h, deg_h):
            pltpu.sync_copy(z128_h.at[pl.ds(0, CHUNK)], rows_v)
            for (o, sz) in pieces:
                pltpu.sync_copy(rows_v.at[pl.ds(0, sz)],
                                acc_sh.at[pl.ds(row0 + o, sz)])
            pltpu.sync_copy(z16_h, dstage_v)
            pltpu.sync_copy(dstage_v, deg_sh.at[pl.ds(row0, SLICE)])

            @pl.when(sid == TILES - 1)
            def _():
                pltpu.sync_copy(rows_v.at[pl.ds(0, TAIL)],
                                acc_sh.at[pl.ds(TILES * SLICE, TAIL)])
                pltpu.sync_copy(dstage_v.at[pl.ds(0, TAIL)],
                                deg_sh.at[pl.ds(TILES * SLICE, TAIL)])

            pltpu.sync_copy(ones_h, ones_v)
            plsc.subcore_barrier()

            def body(i, carry):
                g = i * TILES + sid

                @pl.when(g < NUM_CHUNKS)
                def _():
                    base = g * CHUNK
                    pltpu.sync_copy(ei_h.at[0, pl.ds(base, CHUNK)], src_v)
                    pltpu.sync_copy(ei_h.at[1, pl.ds(base, CHUNK)], dst_v)
                    pltpu.async_copy(x_h.at[src_v], rows_v, sem).wait()
                    pltpu.sync_copy(rows_v, acc_sh.at[dst_v], add=True)
                    pltpu.sync_copy(ones_v, deg_sh.at[dst_v], add=True)

                return carry

            lax.fori_loop(0, CHUNKS_PER_TILE, body, 0)
            plsc.subcore_barrier()
            for (o, sz) in pieces:
                pltpu.sync_copy(acc_sh.at[pl.ds(row0 + o, sz)],
                                rows_v.at[pl.ds(0, sz)])
                pltpu.sync_copy(rows_v.at[pl.ds(0, sz)],
                                agg_h.at[pl.ds(row0 + o, sz)])
            pltpu.sync_copy(deg_sh.at[pl.ds(row0, SLICE)], dstage_v)
            pltpu.sync_copy(dstage_v, deg_h.at[pl.ds(row0, SLICE)])

            @pl.when(sid == TILES - 1)
            def _():
                pltpu.sync_copy(acc_sh.at[pl.ds(TILES * SLICE, TAIL)],
                                rows_v.at[pl.ds(0, TAIL)])
                pltpu.sync_copy(rows_v.at[pl.ds(0, TAIL)],
                                agg_h.at[pl.ds(TILES * SLICE, TAIL)])
                pltpu.sync_copy(deg_sh.at[pl.ds(TILES * SLICE, TAIL)],
                                dstage_v.at[pl.ds(0, TAIL)])
                pltpu.sync_copy(dstage_v.at[pl.ds(0, TAIL)],
                                deg_h.at[pl.ds(TILES * SLICE, TAIL)])

        @pl.when(cid == 0)
        def _():
            run_branch(x0_h, ei0_h, agg0_h, deg0_h)

        @pl.when(cid == 1)
        def _():
            run_branch(x1_h, ei1_h, agg1_h, deg1_h)

    return sc_kernel(x0, ei0, x1, ei1, z128, z16, ones16)


BLK = 1000
NUM_BLK = N // BLK


def _tc_body(agg0_r, deg0_r, b0m_r, agg1_r, deg1_r, b1m_r,
             W0_r, bb0_r, W1_r, bb1_r, f1a_r, f1b_r, f1bias_r, f2w_r, f2b_r,
             out_r, h_r, acc0, cnt0, acc1, cnt1):
    k = pl.program_id(0)

    @pl.when(k == 0)
    def _():
        acc0[...] = jnp.zeros_like(acc0)
        cnt0[...] = jnp.zeros_like(cnt0)
        acc1[...] = jnp.zeros_like(acc1)
        cnt1[...] = jnp.zeros_like(cnt1)

    ones_col = jnp.ones((BLK, 1), jnp.float32)
    gids = lax.broadcasted_iota(jnp.int32, (BLK, NG), 1)

    def branch(agg_r, deg_r, bm_r, W_r, bias_r, acc, cnt):
        deg = jnp.max(deg_r[...], axis=1, keepdims=True)
        inv = 1.0 / jnp.maximum(deg, 1.0)
        h = jnp.maximum(
            jnp.dot(agg_r[...] * inv, W_r[...],
                    preferred_element_type=jnp.float32) + bias_r[...],
            0.0)
        m = (bm_r[...] == gids).astype(jnp.float32)
        acc[...] += lax.dot_general(m, h, (((0,), (0,)), ((), ())),
                                    preferred_element_type=jnp.float32)
        cnt[...] += lax.dot_general(m, ones_col, (((0,), (0,)), ((), ())),
                                    preferred_element_type=jnp.float32)

    branch(agg0_r, deg0_r, b0m_r, W0_r, bb0_r, acc0, cnt0)
    branch(agg1_r, deg1_r, b1m_r, W1_r, bb1_r, acc1, cnt1)

    @pl.when(k == NUM_BLK - 1)
    def _():
        g0 = acc0[...] / jnp.maximum(cnt0[...], 1.0)
        g1 = acc1[...] / jnp.maximum(cnt1[...], 1.0)
        hh = jnp.maximum(
            jnp.dot(g0, f1a_r[...], preferred_element_type=jnp.float32)
            + jnp.dot(g1, f1b_r[...], preferred_element_type=jnp.float32)
            + f1bias_r[...], 0.0)
        h_r[...] = hh
        out_r[...] = jnp.dot(hh, f2w_r[...],
                             preferred_element_type=jnp.float32) + f2b_r[...]


def _tc_head(agg0, deg0, batch0, agg1, deg1, batch1,
             W0, b0, W1, b1, f1a, f1b, f1bias, f2w, f2b):
    row_spec = lambda shape: pl.BlockSpec((BLK,) + shape[1:],
                                          lambda k: (k,) + (0,) * (len(shape) - 1))
    full = lambda shape: pl.BlockSpec(shape, lambda k: (0,) * len(shape))
    return pl.pallas_call(
        _tc_body,
        grid=(NUM_BLK,),
        in_specs=[
            row_spec((N, D)), row_spec((N, DEGW)), row_spec((N, 1)),
            row_spec((N, D)), row_spec((N, DEGW)), row_spec((N, 1)),
            full((D, DIM_EMB)), full((1, DIM_EMB)),
            full((D, DIM_EMB)), full((1, DIM_EMB)),
            full((DIM_EMB, 64)), full((DIM_EMB, 64)), full((1, 64)),
            full((64, DIM_OUT)), full((1, DIM_OUT)),
        ],
        out_specs=[full((NG, DIM_OUT)), full((NG, 64))],
        out_shape=[jax.ShapeDtypeStruct((NG, DIM_OUT), jnp.float32),
                   jax.ShapeDtypeStruct((NG, 64), jnp.float32)],
        scratch_shapes=[
            pltpu.VMEM((NG, DIM_EMB), jnp.float32),
            pltpu.VMEM((NG, 1), jnp.float32),
            pltpu.VMEM((NG, DIM_EMB), jnp.float32),
            pltpu.VMEM((NG, 1), jnp.float32),
        ],
    )(agg0, deg0, batch0, agg1, deg1, batch1,
      W0, b0, W1, b1, f1a, f1b, f1bias, f2w, f2b)


def kernel(x0, edge_index0, batch0, x1, edge_index1, batch1,
           W0, b0, W1, b1, fc1_W, fc1_b, fc2_W, fc2_b):
    z128 = jnp.zeros((SLICE, D), jnp.float32)
    z16 = jnp.zeros((SLICE, DEGW), jnp.float32)
    ones16 = jnp.ones((CHUNK, DEGW), jnp.float32)

    agg0, deg0, agg1, deg1 = _sc_aggregate(
        x0, edge_index0, x1, edge_index1, z128, z16, ones16)

    out, h = _tc_head(
        agg0, deg0, batch0[:, None], agg1, deg1, batch1[:, None],
        W0, b0[None, :], W1, b1[None, :],
        fc1_W[:DIM_EMB], fc1_W[DIM_EMB:], fc1_b[None, :],
        fc2_W, fc2_b[None, :])
    return (out, h)


# speedup vs baseline: 1.2324x; 1.1373x over previous
import functools

import jax
import jax.numpy as jnp
from jax import lax
from jax.experimental import pallas as pl
from jax.experimental.pallas import tpu as pltpu
from jax.experimental.pallas import tpu_sc as plsc

N = 10000
E = 320000
D = 128
NG = 64
DIM_EMB = 128
DIM_OUT = 16

CHUNK = 128
NUM_CHUNKS = E // CHUNK
TILES = 16
CHUNKS_PER_TILE = (NUM_CHUNKS + TILES - 1) // TILES
SLICE = 624
TAIL = N - TILES * SLICE
DEGW = 16


def _sc_aggregate(x0, ei0, x1, ei1, z128, z16, ones16):
    mesh = plsc.VectorSubcoreMesh(core_axis_name="c", subcore_axis_name="s")

    @functools.partial(
        pl.kernel,
        mesh=mesh,
        compiler_params=pltpu.CompilerParams(use_tc_tiling_on_sc=False),
        out_type=[
            jax.ShapeDtypeStruct((N, D), jnp.float32),
            jax.ShapeDtypeStruct((N, DEGW), jnp.float32),
            jax.ShapeDtypeStruct((N, D), jnp.float32),
            jax.ShapeDtypeStruct((N, DEGW), jnp.float32),
        ],
        scratch_types=[
            pltpu.VMEM((2, CHUNK), jnp.int32),
            pltpu.VMEM((CHUNK, D), jnp.float32),
            pltpu.VMEM((CHUNK, DEGW), jnp.float32),
            pltpu.VMEM((SLICE, DEGW), jnp.float32),
            pltpu.VMEM_SHARED((N, D), jnp.float32),
            pltpu.VMEM_SHARED((N, DEGW), jnp.float32),
            pltpu.SemaphoreType.DMA,
        ],
    )
    def sc_kernel(x0_h, ei0_h, x1_h, ei1_h,
                  z128_h, z16_h, ones_h,
                  agg0_h, deg0_h, agg1_h, deg1_h,
                  idx_v, rows_v, ones_v, dstage_v, acc_sh, deg_sh, sem):
        cid = lax.axis_index("c")
        sid = lax.axis_index("s")
        row0 = sid * SLICE
        pieces = []
        off = 0
        while off < SLICE:
            sz = min(CHUNK, SLICE - off)
            pieces.append((off, sz))
            off += sz

        def run_branch(x_h, ei_h, agg_h, deg_h):
            pltpu.sync_copy(z128_h.at[pl.ds(0, CHUNK)], rows_v)
            for (o, sz) in pieces:
                pltpu.sync_copy(rows_v.at[pl.ds(0, sz)],
                                acc_sh.at[pl.ds(row0 + o, sz)])
            pltpu.sync_copy(z16_h, dstage_v)
            pltpu.sync_copy(dstage_v, deg_sh.at[pl.ds(row0, SLICE)])

            @pl.when(sid == TILES - 1)
            def _():
                pltpu.sync_copy(rows_v.at[pl.ds(0, TAIL)],
                                acc_sh.at[pl.ds(TILES * SLICE, TAIL)])
                pltpu.sync_copy(dstage_v.at[pl.ds(0, TAIL)],
                                deg_sh.at[pl.ds(TILES * SLICE, TAIL)])

            pltpu.sync_copy(ones_h, ones_v)
            plsc.subcore_barrier()

            def body(i, carry):
                g = i * TILES + sid

                @pl.when(g < NUM_CHUNKS)
                def _():
                    base = g * CHUNK
                    pltpu.sync_copy(ei_h.at[:, pl.ds(base, CHUNK)], idx_v)
                    pltpu.async_copy(x_h.at[idx_v.at[0]], rows_v, sem).wait()
                    pltpu.sync_copy(rows_v, acc_sh.at[idx_v.at[1]], add=True)
                    pltpu.sync_copy(ones_v, deg_sh.at[idx_v.at[1]], add=True)

                return carry

            lax.fori_loop(0, CHUNKS_PER_TILE, body, 0)
            plsc.subcore_barrier()
            for (o, sz) in pieces:
                pltpu.sync_copy(acc_sh.at[pl.ds(row0 + o, sz)],
                                rows_v.at[pl.ds(0, sz)])
                pltpu.sync_copy(rows_v.at[pl.ds(0, sz)],
                                agg_h.at[pl.ds(row0 + o, sz)])
            pltpu.sync_copy(deg_sh.at[pl.ds(row0, SLICE)], dstage_v)
            pltpu.sync_copy(dstage_v, deg_h.at[pl.ds(row0, SLICE)])

            @pl.when(sid == TILES - 1)
            def _():
                pltpu.sync_copy(acc_sh.at[pl.ds(TILES * SLICE, TAIL)],
                                rows_v.at[pl.ds(0, TAIL)])
                pltpu.sync_copy(rows_v.at[pl.ds(0, TAIL)],
                                agg_h.at[pl.ds(TILES * SLICE, TAIL)])
                pltpu.sync_copy(deg_sh.at[pl.ds(TILES * SLICE, TAIL)],
                                dstage_v.at[pl.ds(0, TAIL)])
                pltpu.sync_copy(dstage_v.at[pl.ds(0, TAIL)],
                                deg_h.at[pl.ds(TILES * SLICE, TAIL)])

        @pl.when(cid == 0)
        def _():
            run_branch(x0_h, ei0_h, agg0_h, deg0_h)

        @pl.when(cid == 1)
        def _():
            run_branch(x1_h, ei1_h, agg1_h, deg1_h)

    return sc_kernel(x0, ei0, x1, ei1, z128, z16, ones16)


BLK = 1000
NUM_BLK = N // BLK


def _tc_body(agg0_r, deg0_r, b0m_r, agg1_r, deg1_r, b1m_r,
             W0_r, bb0_r, W1_r, bb1_r, f1a_r, f1b_r, f1bias_r, f2w_r, f2b_r,
             out_r, h_r, acc0, cnt0, acc1, cnt1):
    k = pl.program_id(0)

    @pl.when(k == 0)
    def _():
        acc0[...] = jnp.zeros_like(acc0)
        cnt0[...] = jnp.zeros_like(cnt0)
        acc1[...] = jnp.zeros_like(acc1)
        cnt1[...] = jnp.zeros_like(cnt1)

    ones_col = jnp.ones((BLK, 1), jnp.float32)
    gids = lax.broadcasted_iota(jnp.int32, (BLK, NG), 1)

    def branch(agg_r, deg_r, bm_r, W_r, bias_r, acc, cnt):
        deg = jnp.max(deg_r[...], axis=1, keepdims=True)
        inv = 1.0 / jnp.maximum(deg, 1.0)
        h = jnp.maximum(
            jnp.dot(agg_r[...] * inv, W_r[...],
                    preferred_element_type=jnp.float32) + bias_r[...],
            0.0)
        m = (bm_r[...] == gids).astype(jnp.float32)
        acc[...] += lax.dot_general(m, h, (((0,), (0,)), ((), ())),
                                    preferred_element_type=jnp.float32)
        cnt[...] += lax.dot_general(m, ones_col, (((0,), (0,)), ((), ())),
                                    preferred_element_type=jnp.float32)

    branch(agg0_r, deg0_r, b0m_r, W0_r, bb0_r, acc0, cnt0)
    branch(agg1_r, deg1_r, b1m_r, W1_r, bb1_r, acc1, cnt1)

    @pl.when(k == NUM_BLK - 1)
    def _():
        g0 = acc0[...] / jnp.maximum(cnt0[...], 1.0)
        g1 = acc1[...] / jnp.maximum(cnt1[...], 1.0)
        hh = jnp.maximum(
            jnp.dot(g0, f1a_r[...], preferred_element_type=jnp.float32)
            + jnp.dot(g1, f1b_r[...], preferred_element_type=jnp.float32)
            + f1bias_r[...], 0.0)
        h_r[...] = hh
        out_r[...] = jnp.dot(hh, f2w_r[...],
                             preferred_element_type=jnp.float32) + f2b_r[...]


def _tc_head(agg0, deg0, batch0, agg1, deg1, batch1,
             W0, b0, W1, b1, f1a, f1b, f1bias, f2w, f2b):
    row_spec = lambda shape: pl.BlockSpec((BLK,) + shape[1:],
                                          lambda k: (k,) + (0,) * (len(shape) - 1))
    full = lambda shape: pl.BlockSpec(shape, lambda k: (0,) * len(shape))
    return pl.pallas_call(
        _tc_body,
        grid=(NUM_BLK,),
        in_specs=[
            row_spec((N, D)), row_spec((N, DEGW)), row_spec((N, 1)),
            row_spec((N, D)), row_spec((N, DEGW)), row_spec((N, 1)),
            full((D, DIM_EMB)), full((1, DIM_EMB)),
            full((D, DIM_EMB)), full((1, DIM_EMB)),
            full((DIM_EMB, 64)), full((DIM_EMB, 64)), full((1, 64)),
            full((64, DIM_OUT)), full((1, DIM_OUT)),
        ],
        out_specs=[full((NG, DIM_OUT)), full((NG, 64))],
        out_shape=[jax.ShapeDtypeStruct((NG, DIM_OUT), jnp.float32),
                   jax.ShapeDtypeStruct((NG, 64), jnp.float32)],
        scratch_shapes=[
            pltpu.VMEM((NG, DIM_EMB), jnp.float32),
            pltpu.VMEM((NG, 1), jnp.float32),
            pltpu.VMEM((NG, DIM_EMB), jnp.float32),
            pltpu.VMEM((NG, 1), jnp.float32),
        ],
    )(agg0, deg0, batch0, agg1, deg1, batch1,
      W0, b0, W1, b1, f1a, f1b, f1bias, f2w, f2b)


def kernel(x0, edge_index0, batch0, x1, edge_index1, batch1,
           W0, b0, W1, b1, fc1_W, fc1_b, fc2_W, fc2_b):
    z128 = jnp.zeros((SLICE, D), jnp.float32)
    z16 = jnp.zeros((SLICE, DEGW), jnp.float32)
    ones16 = jnp.ones((CHUNK, DEGW), jnp.float32)

    agg0, deg0, agg1, deg1 = _sc_aggregate(
        x0, edge_index0, x1, edge_index1, z128, z16, ones16)

    out, h = _tc_head(
        agg0, deg0, batch0[:, None], agg1, deg1, batch1[:, None],
        W0, b0[None, :], W1, b1[None, :],
        fc1_W[:DIM_EMB], fc1_W[DIM_EMB:], fc1_b[None, :],
        fc2_W, fc2_b[None, :])
    return (out, h)


# speedup vs baseline: 1.2595x; 1.0220x over previous
import functools

import jax
import jax.numpy as jnp
from jax import lax
from jax.experimental import pallas as pl
from jax.experimental.pallas import tpu as pltpu
from jax.experimental.pallas import tpu_sc as plsc

N = 10000
E = 320000
D = 128
NG = 64
DIM_EMB = 128
DIM_OUT = 16

CHUNK = 128
NUM_CHUNKS = E // CHUNK
TILES = 16
CHUNKS_PER_TILE = (NUM_CHUNKS + TILES - 1) // TILES
SLICE = 624
TAIL = N - TILES * SLICE
DEGW = 16


def _sc_aggregate(x0, ei0, x1, ei1, z128, z16, ones16):
    mesh = plsc.VectorSubcoreMesh(core_axis_name="c", subcore_axis_name="s")

    @functools.partial(
        pl.kernel,
        mesh=mesh,
        compiler_params=pltpu.CompilerParams(use_tc_tiling_on_sc=False),
        out_type=[
            jax.ShapeDtypeStruct((N, D), jnp.float32),
            jax.ShapeDtypeStruct((N, DEGW), jnp.float32),
            jax.ShapeDtypeStruct((N, D), jnp.float32),
            jax.ShapeDtypeStruct((N, DEGW), jnp.float32),
        ],
        scratch_types=[
            pltpu.VMEM((2, CHUNK), jnp.int32),
            pltpu.VMEM((CHUNK, D), jnp.float32),
            pltpu.VMEM((CHUNK, DEGW), jnp.float32),
            pltpu.VMEM((SLICE, DEGW), jnp.float32),
            pltpu.VMEM_SHARED((N, D), jnp.float32),
            pltpu.VMEM_SHARED((N, DEGW), jnp.float32),
            pltpu.SemaphoreType.DMA,
            pltpu.SemaphoreType.DMA,
        ],
    )
    def sc_kernel(x0_h, ei0_h, x1_h, ei1_h,
                  z128_h, z16_h, ones_h,
                  agg0_h, deg0_h, agg1_h, deg1_h,
                  idx_v, rows_v, ones_v, dstage_v, acc_sh, deg_sh, sem,
                  sem_s):
        cid = lax.axis_index("c")
        sid = lax.axis_index("s")
        row0 = sid * SLICE
        pieces = []
        off = 0
        while off < SLICE:
            sz = min(CHUNK, SLICE - off)
            pieces.append((off, sz))
            off += sz

        def run_branch(x_h, ei_h, agg_h, deg_h):
            pltpu.sync_copy(z128_h.at[pl.ds(0, CHUNK)], rows_v)
            for (o, sz) in pieces:
                pltpu.sync_copy(rows_v.at[pl.ds(0, sz)],
                                acc_sh.at[pl.ds(row0 + o, sz)])
            pltpu.sync_copy(z16_h, dstage_v)
            pltpu.sync_copy(dstage_v, deg_sh.at[pl.ds(row0, SLICE)])

            @pl.when(sid == TILES - 1)
            def _():
                pltpu.sync_copy(rows_v.at[pl.ds(0, TAIL)],
                                acc_sh.at[pl.ds(TILES * SLICE, TAIL)])
                pltpu.sync_copy(dstage_v.at[pl.ds(0, TAIL)],
                                deg_sh.at[pl.ds(TILES * SLICE, TAIL)])

            pltpu.sync_copy(ones_h, ones_v)
            plsc.subcore_barrier()

            def body(i, carry):
                g = i * TILES + sid

                @pl.when(g < NUM_CHUNKS)
                def _():
                    base = g * CHUNK
                    pltpu.sync_copy(ei_h.at[:, pl.ds(base, CHUNK)], idx_v)
                    pltpu.async_copy(x_h.at[idx_v.at[0]], rows_v, sem).wait()
                    s0 = pltpu.async_copy(rows_v, acc_sh.at[idx_v.at[1]],
                                          sem_s, add=True)
                    s1 = pltpu.async_copy(ones_v, deg_sh.at[idx_v.at[1]],
                                          sem_s, add=True)
                    s0.wait()
                    s1.wait()

                return carry

            lax.fori_loop(0, CHUNKS_PER_TILE, body, 0)
            plsc.subcore_barrier()
            for (o, sz) in pieces:
                pltpu.sync_copy(acc_sh.at[pl.ds(row0 + o, sz)],
                                rows_v.at[pl.ds(0, sz)])
                pltpu.sync_copy(rows_v.at[pl.ds(0, sz)],
                                agg_h.at[pl.ds(row0 + o, sz)])
            pltpu.sync_copy(deg_sh.at[pl.ds(row0, SLICE)], dstage_v)
            pltpu.sync_copy(dstage_v, deg_h.at[pl.ds(row0, SLICE)])

            @pl.when(sid == TILES - 1)
            def _():
                pltpu.sync_copy(acc_sh.at[pl.ds(TILES * SLICE, TAIL)],
                                rows_v.at[pl.ds(0, TAIL)])
                pltpu.sync_copy(rows_v.at[pl.ds(0, TAIL)],
                                agg_h.at[pl.ds(TILES * SLICE, TAIL)])
                pltpu.sync_copy(deg_sh.at[pl.ds(TILES * SLICE, TAIL)],
                                dstage_v.at[pl.ds(0, TAIL)])
                pltpu.sync_copy(dstage_v.at[pl.ds(0, TAIL)],
                                deg_h.at[pl.ds(TILES * SLICE, TAIL)])

        @pl.when(cid == 0)
        def _():
            run_branch(x0_h, ei0_h, agg0_h, deg0_h)

        @pl.when(cid == 1)
        def _():
            run_branch(x1_h, ei1_h, agg1_h, deg1_h)

    return sc_kernel(x0, ei0, x1, ei1, z128, z16, ones16)


BLK = 1000
NUM_BLK = N // BLK


def _tc_body(agg0_r, deg0_r, b0m_r, agg1_r, deg1_r, b1m_r,
             W0_r, bb0_r, W1_r, bb1_r, f1a_r, f1b_r, f1bias_r, f2w_r, f2b_r,
             out_r, h_r, acc0, cnt0, acc1, cnt1):
    k = pl.program_id(0)

    @pl.when(k == 0)
    def _():
        acc0[...] = jnp.zeros_like(acc0)
        cnt0[...] = jnp.zeros_like(cnt0)
        acc1[...] = jnp.zeros_like(acc1)
        cnt1[...] = jnp.zeros_like(cnt1)

    ones_col = jnp.ones((BLK, 1), jnp.float32)
    gids = lax.broadcasted_iota(jnp.int32, (BLK, NG), 1)

    def branch(agg_r, deg_r, bm_r, W_r, bias_r, acc, cnt):
        deg = jnp.max(deg_r[...], axis=1, keepdims=True)
        inv = 1.0 / jnp.maximum(deg, 1.0)
        h = jnp.maximum(
            jnp.dot(agg_r[...] * inv, W_r[...],
                    preferred_element_type=jnp.float32) + bias_r[...],
            0.0)
        m = (bm_r[...] == gids).astype(jnp.float32)
        acc[...] += lax.dot_general(m, h, (((0,), (0,)), ((), ())),
                                    preferred_element_type=jnp.float32)
        cnt[...] += lax.dot_general(m, ones_col, (((0,), (0,)), ((), ())),
                                    preferred_element_type=jnp.float32)

    branch(agg0_r, deg0_r, b0m_r, W0_r, bb0_r, acc0, cnt0)
    branch(agg1_r, deg1_r, b1m_r, W1_r, bb1_r, acc1, cnt1)

    @pl.when(k == NUM_BLK - 1)
    def _():
        g0 = acc0[...] / jnp.maximum(cnt0[...], 1.0)
        g1 = acc1[...] / jnp.maximum(cnt1[...], 1.0)
        hh = jnp.maximum(
            jnp.dot(g0, f1a_r[...], preferred_element_type=jnp.float32)
            + jnp.dot(g1, f1b_r[...], preferred_element_type=jnp.float32)
            + f1bias_r[...], 0.0)
        h_r[...] = hh
        out_r[...] = jnp.dot(hh, f2w_r[...],
                             preferred_element_type=jnp.float32) + f2b_r[...]


def _tc_head(agg0, deg0, batch0, agg1, deg1, batch1,
             W0, b0, W1, b1, f1a, f1b, f1bias, f2w, f2b):
    row_spec = lambda shape: pl.BlockSpec((BLK,) + shape[1:],
                                          lambda k: (k,) + (0,) * (len(shape) - 1))
    full = lambda shape: pl.BlockSpec(shape, lambda k: (0,) * len(shape))
    return pl.pallas_call(
        _tc_body,
        grid=(NUM_BLK,),
        in_specs=[
            row_spec((N, D)), row_spec((N, DEGW)), row_spec((N, 1)),
            row_spec((N, D)), row_spec((N, DEGW)), row_spec((N, 1)),
            full((D, DIM_EMB)), full((1, DIM_EMB)),
            full((D, DIM_EMB)), full((1, DIM_EMB)),
            full((DIM_EMB, 64)), full((DIM_EMB, 64)), full((1, 64)),
            full((64, DIM_OUT)), full((1, DIM_OUT)),
        ],
        out_specs=[full((NG, DIM_OUT)), full((NG, 64))],
        out_shape=[jax.ShapeDtypeStruct((NG, DIM_OUT), jnp.float32),
                   jax.ShapeDtypeStruct((NG, 64), jnp.float32)],
        scratch_shapes=[
            pltpu.VMEM((NG, DIM_EMB), jnp.float32),
            pltpu.VMEM((NG, 1), jnp.float32),
            pltpu.VMEM((NG, DIM_EMB), jnp.float32),
            pltpu.VMEM((NG, 1), jnp.float32),
        ],
    )(agg0, deg0, batch0, agg1, deg1, batch1,
      W0, b0, W1, b1, f1a, f1b, f1bias, f2w, f2b)


def kernel(x0, edge_index0, batch0, x1, edge_index1, batch1,
           W0, b0, W1, b1, fc1_W, fc1_b, fc2_W, fc2_b):
    z128 = jnp.zeros((SLICE, D), jnp.float32)
    z16 = jnp.zeros((SLICE, DEGW), jnp.float32)
    ones16 = jnp.ones((CHUNK, DEGW), jnp.float32)

    agg0, deg0, agg1, deg1 = _sc_aggregate(
        x0, edge_index0, x1, edge_index1, z128, z16, ones16)

    out, h = _tc_head(
        agg0, deg0, batch0[:, None], agg1, deg1, batch1[:, None],
        W0, b0[None, :], W1, b1[None, :],
        fc1_W[:DIM_EMB], fc1_W[DIM_EMB:], fc1_b[None, :],
        fc2_W, fc2_b[None, :])
    return (out, h)


# speedup vs baseline: 1.4846x; 1.1787x over previous
import functools

import jax
import jax.numpy as jnp
from jax import lax
from jax.experimental import pallas as pl
from jax.experimental.pallas import tpu as pltpu
from jax.experimental.pallas import tpu_sc as plsc

N = 10000
E = 320000
D = 128
NG = 64
DIM_EMB = 128
DIM_OUT = 16

CHUNK = 128
NUM_CHUNKS = E // CHUNK
TILES = 16
CHUNKS_PER_TILE = (NUM_CHUNKS + TILES - 1) // TILES
SLICE = 624
TAIL = N - TILES * SLICE
DEGW = 16


def _sc_aggregate(x0, ei0, x1, ei1, z128, z16, ones16):
    mesh = plsc.VectorSubcoreMesh(core_axis_name="c", subcore_axis_name="s")

    @functools.partial(
        pl.kernel,
        mesh=mesh,
        compiler_params=pltpu.CompilerParams(use_tc_tiling_on_sc=False),
        out_type=[
            jax.ShapeDtypeStruct((N, D), jnp.float32),
            jax.ShapeDtypeStruct((N, DEGW), jnp.float32),
            jax.ShapeDtypeStruct((N, D), jnp.float32),
            jax.ShapeDtypeStruct((N, DEGW), jnp.float32),
        ],
        scratch_types=[
            pltpu.VMEM((2, CHUNK), jnp.int32),
            pltpu.VMEM((2, CHUNK), jnp.int32),
            pltpu.VMEM((CHUNK, D), jnp.float32),
            pltpu.VMEM((CHUNK, DEGW), jnp.float32),
            pltpu.VMEM((SLICE, DEGW), jnp.float32),
            pltpu.VMEM_SHARED((N, D), jnp.float32),
            pltpu.VMEM_SHARED((N, DEGW), jnp.float32),
            pltpu.SemaphoreType.DMA,
            pltpu.SemaphoreType.DMA,
            pltpu.SemaphoreType.DMA,
            pltpu.SemaphoreType.DMA,
        ],
    )
    def sc_kernel(x0_h, ei0_h, x1_h, ei1_h,
                  z128_h, z16_h, ones_h,
                  agg0_h, deg0_h, agg1_h, deg1_h,
                  idx_a, idx_b, rows_v, ones_v, dstage_v, acc_sh, deg_sh,
                  sem, sem_s, sem_ib, sem_g):
        cid = lax.axis_index("c")
        sid = lax.axis_index("s")
        row0 = sid * SLICE
        pieces = []
        off = 0
        while off < SLICE:
            sz = min(CHUNK, SLICE - off)
            pieces.append((off, sz))
            off += sz

        def run_branch(x_h, ei_h, agg_h, deg_h):
            pltpu.sync_copy(z128_h.at[pl.ds(0, CHUNK)], rows_v)
            for (o, sz) in pieces:
                pltpu.sync_copy(rows_v.at[pl.ds(0, sz)],
                                acc_sh.at[pl.ds(row0 + o, sz)])
            pltpu.sync_copy(z16_h, dstage_v)
            pltpu.sync_copy(dstage_v, deg_sh.at[pl.ds(row0, SLICE)])

            @pl.when(sid == TILES - 1)
            def _():
                pltpu.sync_copy(rows_v.at[pl.ds(0, TAIL)],
                                acc_sh.at[pl.ds(TILES * SLICE, TAIL)])
                pltpu.sync_copy(dstage_v.at[pl.ds(0, TAIL)],
                                deg_sh.at[pl.ds(TILES * SLICE, TAIL)])

            pltpu.sync_copy(ones_h, ones_v)
            plsc.subcore_barrier()

            def fetch_idx(g, buf, fsem):
                return pltpu.async_copy(
                    ei_h.at[:, pl.ds(g * CHUNK, CHUNK)], buf, fsem)

            def process(g, buf):
                pltpu.async_copy(x_h.at[buf.at[0]], rows_v, sem_g).wait()
                s0 = pltpu.async_copy(rows_v, acc_sh.at[buf.at[1]],
                                      sem_s, add=True)
                s1 = pltpu.async_copy(ones_v, deg_sh.at[buf.at[1]],
                                      sem_s, add=True)
                s0.wait()
                s1.wait()

            @pl.when(sid < NUM_CHUNKS)
            def _():
                fetch_idx(sid, idx_a, sem)

            def body(i, carry):
                ga = (2 * i) * TILES + sid
                gb = ga + TILES
                ga2 = ga + 2 * TILES

                @pl.when(gb < NUM_CHUNKS)
                def _():
                    fetch_idx(gb, idx_b, sem_ib)

                @pl.when(ga < NUM_CHUNKS)
                def _():
                    pltpu.make_async_copy(
                        ei_h.at[:, pl.ds(ga * CHUNK, CHUNK)], idx_a,
                        sem).wait()
                    process(ga, idx_a)

                @pl.when(ga2 < NUM_CHUNKS)
                def _():
                    fetch_idx(ga2, idx_a, sem)

                @pl.when(gb < NUM_CHUNKS)
                def _():
                    pltpu.make_async_copy(
                        ei_h.at[:, pl.ds(gb * CHUNK, CHUNK)], idx_b,
                        sem_ib).wait()
                    process(gb, idx_b)

                return carry

            lax.fori_loop(0, (CHUNKS_PER_TILE + 1) // 2, body, 0)
            plsc.subcore_barrier()
            for (o, sz) in pieces:
                pltpu.sync_copy(acc_sh.at[pl.ds(row0 + o, sz)],
                                rows_v.at[pl.ds(0, sz)])
                pltpu.sync_copy(rows_v.at[pl.ds(0, sz)],
                                agg_h.at[pl.ds(row0 + o, sz)])
            pltpu.sync_copy(deg_sh.at[pl.ds(row0, SLICE)], dstage_v)
            pltpu.sync_copy(dstage_v, deg_h.at[pl.ds(row0, SLICE)])

            @pl.when(sid == TILES - 1)
            def _():
                pltpu.sync_copy(acc_sh.at[pl.ds(TILES * SLICE, TAIL)],
                                rows_v.at[pl.ds(0, TAIL)])
                pltpu.sync_copy(rows_v.at[pl.ds(0, TAIL)],
                                agg_h.at[pl.ds(TILES * SLICE, TAIL)])
                pltpu.sync_copy(deg_sh.at[pl.ds(TILES * SLICE, TAIL)],
                                dstage_v.at[pl.ds(0, TAIL)])
                pltpu.sync_copy(dstage_v.at[pl.ds(0, TAIL)],
                                deg_h.at[pl.ds(TILES * SLICE, TAIL)])

        @pl.when(cid == 0)
        def _():
            run_branch(x0_h, ei0_h, agg0_h, deg0_h)

        @pl.when(cid == 1)
        def _():
            run_branch(x1_h, ei1_h, agg1_h, deg1_h)

    return sc_kernel(x0, ei0, x1, ei1, z128, z16, ones16)


BLK = 1000
NUM_BLK = N // BLK


def _tc_body(agg0_r, deg0_r, b0m_r, agg1_r, deg1_r, b1m_r,
             W0_r, bb0_r, W1_r, bb1_r, f1a_r, f1b_r, f1bias_r, f2w_r, f2b_r,
             out_r, h_r, acc0, cnt0, acc1, cnt1):
    k = pl.program_id(0)

    @pl.when(k == 0)
    def _():
        acc0[...] = jnp.zeros_like(acc0)
        cnt0[...] = jnp.zeros_like(cnt0)
        acc1[...] = jnp.zeros_like(acc1)
        cnt1[...] = jnp.zeros_like(cnt1)

    ones_col = jnp.ones((BLK, 1), jnp.float32)
    gids = lax.broadcasted_iota(jnp.int32, (BLK, NG), 1)

    def branch(agg_r, deg_r, bm_r, W_r, bias_r, acc, cnt):
        deg = jnp.max(deg_r[...], axis=1, keepdims=True)
        inv = 1.0 / jnp.maximum(deg, 1.0)
        h = jnp.maximum(
            jnp.dot(agg_r[...] * inv, W_r[...],
                    preferred_element_type=jnp.float32) + bias_r[...],
            0.0)
        m = (bm_r[...] == gids).astype(jnp.float32)
        acc[...] += lax.dot_general(m, h, (((0,), (0,)), ((), ())),
                                    preferred_element_type=jnp.float32)
        cnt[...] += lax.dot_general(m, ones_col, (((0,), (0,)), ((), ())),
                                    preferred_element_type=jnp.float32)

    branch(agg0_r, deg0_r, b0m_r, W0_r, bb0_r, acc0, cnt0)
    branch(agg1_r, deg1_r, b1m_r, W1_r, bb1_r, acc1, cnt1)

    @pl.when(k == NUM_BLK - 1)
    def _():
        g0 = acc0[...] / jnp.maximum(cnt0[...], 1.0)
        g1 = acc1[...] / jnp.maximum(cnt1[...], 1.0)
        hh = jnp.maximum(
            jnp.dot(g0, f1a_r[...], preferred_element_type=jnp.float32)
            + jnp.dot(g1, f1b_r[...], preferred_element_type=jnp.float32)
            + f1bias_r[...], 0.0)
        h_r[...] = hh
        out_r[...] = jnp.dot(hh, f2w_r[...],
                             preferred_element_type=jnp.float32) + f2b_r[...]


def _tc_head(agg0, deg0, batch0, agg1, deg1, batch1,
             W0, b0, W1, b1, f1a, f1b, f1bias, f2w, f2b):
    row_spec = lambda shape: pl.BlockSpec((BLK,) + shape[1:],
                                          lambda k: (k,) + (0,) * (len(shape) - 1))
    full = lambda shape: pl.BlockSpec(shape, lambda k: (0,) * len(shape))
    return pl.pallas_call(
        _tc_body,
        grid=(NUM_BLK,),
        in_specs=[
            row_spec((N, D)), row_spec((N, DEGW)), row_spec((N, 1)),
            row_spec((N, D)), row_spec((N, DEGW)), row_spec((N, 1)),
            full((D, DIM_EMB)), full((1, DIM_EMB)),
            full((D, DIM_EMB)), full((1, DIM_EMB)),
            full((DIM_EMB, 64)), full((DIM_EMB, 64)), full((1, 64)),
            full((64, DIM_OUT)), full((1, DIM_OUT)),
        ],
        out_specs=[full((NG, DIM_OUT)), full((NG, 64))],
        out_shape=[jax.ShapeDtypeStruct((NG, DIM_OUT), jnp.float32),
                   jax.ShapeDtypeStruct((NG, 64), jnp.float32)],
        scratch_shapes=[
            pltpu.VMEM((NG, DIM_EMB), jnp.float32),
            pltpu.VMEM((NG, 1), jnp.float32),
            pltpu.VMEM((NG, DIM_EMB), jnp.float32),
            pltpu.VMEM((NG, 1), jnp.float32),
        ],
    )(agg0, deg0, batch0, agg1, deg1, batch1,
      W0, b0, W1, b1, f1a, f1b, f1bias, f2w, f2b)


def kernel(x0, edge_index0, batch0, x1, edge_index1, batch1,
           W0, b0, W1, b1, fc1_W, fc1_b, fc2_W, fc2_b):
    z128 = jnp.zeros((SLICE, D), jnp.float32)
    z16 = jnp.zeros((SLICE, DEGW), jnp.float32)
    ones16 = jnp.ones((CHUNK, DEGW), jnp.float32)

    agg0, deg0, agg1, deg1 = _sc_aggregate(
        x0, edge_index0, x1, edge_index1, z128, z16, ones16)

    out, h = _tc_head(
        agg0, deg0, batch0[:, None], agg1, deg1, batch1[:, None],
        W0, b0[None, :], W1, b1[None, :],
        fc1_W[:DIM_EMB], fc1_W[DIM_EMB:], fc1_b[None, :],
        fc2_W, fc2_b[None, :])
    return (out, h)


# speedup vs baseline: 1.9713x; 1.3278x over previous
import functools

import jax
import jax.numpy as jnp
from jax import lax
from jax.experimental import pallas as pl
from jax.experimental.pallas import tpu as pltpu
from jax.experimental.pallas import tpu_sc as plsc

N = 10000
E = 320000
D = 128
NG = 64
DIM_EMB = 128
DIM_OUT = 16

CHUNK = 128
NUM_CHUNKS = E // CHUNK
TILES = 16
CHUNKS_PER_TILE = (NUM_CHUNKS + TILES - 1) // TILES
SLICE = 624
TAIL = N - TILES * SLICE
DEGW = 16


def _sc_aggregate(x0, ei0, x1, ei1, z128, z16, ones16):
    mesh = plsc.VectorSubcoreMesh(core_axis_name="c", subcore_axis_name="s")

    @functools.partial(
        pl.kernel,
        mesh=mesh,
        compiler_params=pltpu.CompilerParams(use_tc_tiling_on_sc=False),
        out_type=[
            jax.ShapeDtypeStruct((N, D), jnp.float32),
            jax.ShapeDtypeStruct((N, DEGW), jnp.float32),
            jax.ShapeDtypeStruct((N, D), jnp.float32),
            jax.ShapeDtypeStruct((N, DEGW), jnp.float32),
        ],
        scratch_types=[
            pltpu.VMEM((2, CHUNK), jnp.int32),
            pltpu.VMEM((2, CHUNK), jnp.int32),
            pltpu.VMEM((CHUNK, D), jnp.float32),
            pltpu.VMEM((CHUNK, D), jnp.float32),
            pltpu.VMEM((CHUNK, DEGW), jnp.float32),
            pltpu.VMEM_SHARED((N, D), jnp.float32),
            pltpu.VMEM_SHARED((N, DEGW), jnp.float32),
            pltpu.SemaphoreType.DMA,
            pltpu.SemaphoreType.DMA,
            pltpu.SemaphoreType.DMA,
            pltpu.SemaphoreType.DMA,
            pltpu.SemaphoreType.DMA,
        ],
    )
    def sc_kernel(x0_h, ei0_h, x1_h, ei1_h,
                  z128_h, z16_h, ones_h,
                  agg0_h, deg0_h, agg1_h, deg1_h,
                  idx_a, idx_b, rows_va, rows_vb, ones_v,
                  acc_sh, deg_sh, sem, sem_s, sem_ib, sem_g, sem_gb):
        cid = lax.axis_index("c")
        sid = lax.axis_index("s")
        row0 = sid * SLICE
        pieces = []
        off = 0
        while off < SLICE:
            sz = min(CHUNK, SLICE - off)
            pieces.append((off, sz))
            off += sz

        def run_branch(x_h, ei_h, agg_h, deg_h):
            pltpu.sync_copy(z128_h.at[pl.ds(0, CHUNK)], rows_va)
            pltpu.sync_copy(z16_h.at[pl.ds(0, CHUNK)], ones_v)
            for (o, sz) in pieces:
                pltpu.sync_copy(rows_va.at[pl.ds(0, sz)],
                                acc_sh.at[pl.ds(row0 + o, sz)])
                pltpu.sync_copy(ones_v.at[pl.ds(0, sz)],
                                deg_sh.at[pl.ds(row0 + o, sz)])

            @pl.when(sid == TILES - 1)
            def _():
                pltpu.sync_copy(rows_va.at[pl.ds(0, TAIL)],
                                acc_sh.at[pl.ds(TILES * SLICE, TAIL)])
                pltpu.sync_copy(ones_v.at[pl.ds(0, TAIL)],
                                deg_sh.at[pl.ds(TILES * SLICE, TAIL)])

            pltpu.sync_copy(ones_h, ones_v)
            plsc.subcore_barrier()

            def fetch_idx(g, buf, fsem):
                return pltpu.async_copy(
                    ei_h.at[:, pl.ds(g * CHUNK, CHUNK)], buf, fsem)

            def start_gather(buf, rows, gsem):
                return pltpu.async_copy(x_h.at[buf.at[0]], rows, gsem)

            def wait_gather(buf, rows, gsem):
                pltpu.make_async_copy(x_h.at[buf.at[0]], rows, gsem).wait()

            def scatter(buf, rows):
                s0 = pltpu.async_copy(rows, acc_sh.at[buf.at[1]],
                                      sem_s, add=True)
                s1 = pltpu.async_copy(ones_v, deg_sh.at[buf.at[1]],
                                      sem_s, add=True)
                s0.wait()
                s1.wait()

            def wait_idx(g, buf, fsem):
                pltpu.make_async_copy(
                    ei_h.at[:, pl.ds(g * CHUNK, CHUNK)], buf, fsem).wait()

            @pl.when(sid < NUM_CHUNKS)
            def _():
                fetch_idx(sid, idx_a, sem)

            @pl.when(sid + TILES < NUM_CHUNKS)
            def _():
                fetch_idx(sid + TILES, idx_b, sem_ib)

            @pl.when(sid < NUM_CHUNKS)
            def _():
                wait_idx(sid, idx_a, sem)
                start_gather(idx_a, rows_va, sem_g)

            def body(i, carry):
                ga = (2 * i) * TILES + sid
                gb = ga + TILES
                ga2 = ga + 2 * TILES
                gb2 = ga + 3 * TILES

                @pl.when(ga < NUM_CHUNKS)
                def _():
                    wait_gather(idx_a, rows_va, sem_g)

                    @pl.when(gb < NUM_CHUNKS)
                    def _():
                        wait_idx(gb, idx_b, sem_ib)
                        start_gather(idx_b, rows_vb, sem_gb)

                    scatter(idx_a, rows_va)

                @pl.when(ga2 < NUM_CHUNKS)
                def _():
                    fetch_idx(ga2, idx_a, sem)

                @pl.when(gb < NUM_CHUNKS)
                def _():
                    wait_gather(idx_b, rows_vb, sem_gb)

                    @pl.when(ga2 < NUM_CHUNKS)
                    def _():
                        wait_idx(ga2, idx_a, sem)
                        start_gather(idx_a, rows_va, sem_g)

                    scatter(idx_b, rows_vb)

                @pl.when(gb2 < NUM_CHUNKS)
                def _():
                    fetch_idx(gb2, idx_b, sem_ib)

                return carry

            lax.fori_loop(0, (CHUNKS_PER_TILE + 1) // 2, body, 0)
            plsc.subcore_barrier()
            for (o, sz) in pieces:
                pltpu.sync_copy(acc_sh.at[pl.ds(row0 + o, sz)],
                                rows_va.at[pl.ds(0, sz)])
                pltpu.sync_copy(rows_va.at[pl.ds(0, sz)],
                                agg_h.at[pl.ds(row0 + o, sz)])
                pltpu.sync_copy(deg_sh.at[pl.ds(row0 + o, sz)],
                                ones_v.at[pl.ds(0, sz)])
                pltpu.sync_copy(ones_v.at[pl.ds(0, sz)],
                                deg_h.at[pl.ds(row0 + o, sz)])

            @pl.when(sid == TILES - 1)
            def _():
                pltpu.sync_copy(acc_sh.at[pl.ds(TILES * SLICE, TAIL)],
                                rows_va.at[pl.ds(0, TAIL)])
                pltpu.sync_copy(rows_va.at[pl.ds(0, TAIL)],
                                agg_h.at[pl.ds(TILES * SLICE, TAIL)])
                pltpu.sync_copy(deg_sh.at[pl.ds(TILES * SLICE, TAIL)],
                                ones_v.at[pl.ds(0, TAIL)])
                pltpu.sync_copy(ones_v.at[pl.ds(0, TAIL)],
                                deg_h.at[pl.ds(TILES * SLICE, TAIL)])

        @pl.when(cid == 0)
        def _():
            run_branch(x0_h, ei0_h, agg0_h, deg0_h)

        @pl.when(cid == 1)
        def _():
            run_branch(x1_h, ei1_h, agg1_h, deg1_h)

    return sc_kernel(x0, ei0, x1, ei1, z128, z16, ones16)


BLK = 1000
NUM_BLK = N // BLK


def _tc_body(agg0_r, deg0_r, b0m_r, agg1_r, deg1_r, b1m_r,
             W0_r, bb0_r, W1_r, bb1_r, f1a_r, f1b_r, f1bias_r, f2w_r, f2b_r,
             out_r, h_r, acc0, cnt0, acc1, cnt1):
    k = pl.program_id(0)

    @pl.when(k == 0)
    def _():
        acc0[...] = jnp.zeros_like(acc0)
        cnt0[...] = jnp.zeros_like(cnt0)
        acc1[...] = jnp.zeros_like(acc1)
        cnt1[...] = jnp.zeros_like(cnt1)

    ones_col = jnp.ones((BLK, 1), jnp.float32)
    gids = lax.broadcasted_iota(jnp.int32, (BLK, NG), 1)

    def branch(agg_r, deg_r, bm_r, W_r, bias_r, acc, cnt):
        deg = jnp.max(deg_r[...], axis=1, keepdims=True)
        inv = 1.0 / jnp.maximum(deg, 1.0)
        h = jnp.maximum(
            jnp.dot(agg_r[...] * inv, W_r[...],
                    preferred_element_type=jnp.float32) + bias_r[...],
            0.0)
        m = (bm_r[...] == gids).astype(jnp.float32)
        acc[...] += lax.dot_general(m, h, (((0,), (0,)), ((), ())),
                                    preferred_element_type=jnp.float32)
        cnt[...] += lax.dot_general(m, ones_col, (((0,), (0,)), ((), ())),
                                    preferred_element_type=jnp.float32)

    branch(agg0_r, deg0_r, b0m_r, W0_r, bb0_r, acc0, cnt0)
    branch(agg1_r, deg1_r, b1m_r, W1_r, bb1_r, acc1, cnt1)

    @pl.when(k == NUM_BLK - 1)
    def _():
        g0 = acc0[...] / jnp.maximum(cnt0[...], 1.0)
        g1 = acc1[...] / jnp.maximum(cnt1[...], 1.0)
        hh = jnp.maximum(
            jnp.dot(g0, f1a_r[...], preferred_element_type=jnp.float32)
            + jnp.dot(g1, f1b_r[...], preferred_element_type=jnp.float32)
            + f1bias_r[...], 0.0)
        h_r[...] = hh
        out_r[...] = jnp.dot(hh, f2w_r[...],
                             preferred_element_type=jnp.float32) + f2b_r[...]


def _tc_head(agg0, deg0, batch0, agg1, deg1, batch1,
             W0, b0, W1, b1, f1a, f1b, f1bias, f2w, f2b):
    row_spec = lambda shape: pl.BlockSpec((BLK,) + shape[1:],
                                          lambda k: (k,) + (0,) * (len(shape) - 1))
    full = lambda shape: pl.BlockSpec(shape, lambda k: (0,) * len(shape))
    return pl.pallas_call(
        _tc_body,
        grid=(NUM_BLK,),
        in_specs=[
            row_spec((N, D)), row_spec((N, DEGW)), row_spec((N, 1)),
            row_spec((N, D)), row_spec((N, DEGW)), row_spec((N, 1)),
            full((D, DIM_EMB)), full((1, DIM_EMB)),
            full((D, DIM_EMB)), full((1, DIM_EMB)),
            full((DIM_EMB, 64)), full((DIM_EMB, 64)), full((1, 64)),
            full((64, DIM_OUT)), full((1, DIM_OUT)),
        ],
        out_specs=[full((NG, DIM_OUT)), full((NG, 64))],
        out_shape=[jax.ShapeDtypeStruct((NG, DIM_OUT), jnp.float32),
                   jax.ShapeDtypeStruct((NG, 64), jnp.float32)],
        scratch_shapes=[
            pltpu.VMEM((NG, DIM_EMB), jnp.float32),
            pltpu.VMEM((NG, 1), jnp.float32),
            pltpu.VMEM((NG, DIM_EMB), jnp.float32),
            pltpu.VMEM((NG, 1), jnp.float32),
        ],
    )(agg0, deg0, batch0, agg1, deg1, batch1,
      W0, b0, W1, b1, f1a, f1b, f1bias, f2w, f2b)


def kernel(x0, edge_index0, batch0, x1, edge_index1, batch1,
           W0, b0, W1, b1, fc1_W, fc1_b, fc2_W, fc2_b):
    z128 = jnp.zeros((SLICE, D), jnp.float32)
    z16 = jnp.zeros((SLICE, DEGW), jnp.float32)
    ones16 = jnp.ones((CHUNK, DEGW), jnp.float32)

    agg0, deg0, agg1, deg1 = _sc_aggregate(
        x0, edge_index0, x1, edge_index1, z128, z16, ones16)

    out, h = _tc_head(
        agg0, deg0, batch0[:, None], agg1, deg1, batch1[:, None],
        W0, b0[None, :], W1, b1[None, :],
        fc1_W[:DIM_EMB], fc1_W[DIM_EMB:], fc1_b[None, :],
        fc2_W, fc2_b[None, :])
    return (out, h)
